# baseline (speedup 1.0000x reference)
import functools

import jax
import jax.numpy as jnp
from jax import lax
from jax.experimental import pallas as pl
from jax.experimental.pallas import tpu as pltpu
from jax.experimental.pallas import tpu_sc as plsc

VOCAB = 1000000
DIM = 64
BATCH = 16384

NC = 2
NS = 16
L = 16
NW = NC * NS
BPW = BATCH // NW
CH = 128
NCH = BPW // CH


def _glove_body(i_hbm, j_hbm, ui_hbm, uj_hbm, bi_hbm, bj_hbm, out_hbm,
                idx_i, idx_j, rows_i, rows_j, b_i, b_j, out_v, sem):
    cid = lax.axis_index("c")
    sid = lax.axis_index("s")
    wid = sid * NC + cid
    base = wid * BPW

    pltpu.sync_copy(i_hbm.at[pl.ds(base, BPW)], idx_i)
    pltpu.sync_copy(j_hbm.at[pl.ds(base, BPW)], idx_j)

    copies = []
    for c in range(NCH):
        s = pl.ds(c * CH, CH)
        copies.append(pltpu.async_copy(ui_hbm.at[idx_i.at[s]], rows_i.at[s], sem))
        copies.append(pltpu.async_copy(uj_hbm.at[idx_j.at[s]], rows_j.at[s], sem))
        copies.append(pltpu.async_copy(bi_hbm.at[idx_i.at[s]], b_i.at[s], sem))
        copies.append(pltpu.async_copy(bj_hbm.at[idx_j.at[s]], b_j.at[s], sem))
    for cp in copies:
        cp.wait()

    lanes = lax.iota(jnp.int32, L)

    def group(g, carry):
        acc = b_i[pl.ds(g * L, L)] + b_j[pl.ds(g * L, L)]
        for l in range(L):
            r = g * L + l
            partial = rows_i[r, pl.ds(0, L)] * rows_j[r, pl.ds(0, L)]
            for k in range(1, DIM // L):
                partial = partial + rows_i[r, pl.ds(k * L, L)] * rows_j[r, pl.ds(k * L, L)]
            acc = jnp.where(lanes == l, acc + jnp.sum(partial), acc)
        out_v[pl.ds(g * L, L)] = acc
        return carry

    lax.fori_loop(0, BPW // L, group, 0)
    pltpu.sync_copy(out_v, out_hbm.at[pl.ds(base, BPW)])


_glove_call = pl.kernel(
    _glove_body,
    out_type=jax.ShapeDtypeStruct((BATCH,), jnp.float32),
    mesh=plsc.VectorSubcoreMesh(
        core_axis_name="c", subcore_axis_name="s", num_cores=NC, num_subcores=NS
    ),
    compiler_params=pltpu.CompilerParams(
        needs_layout_passes=False, use_tc_tiling_on_sc=False
    ),
    scratch_types=[
        pltpu.VMEM((BPW,), jnp.int32),
        pltpu.VMEM((BPW,), jnp.int32),
        pltpu.VMEM((BPW, DIM), jnp.float32),
        pltpu.VMEM((BPW, DIM), jnp.float32),
        pltpu.VMEM((BPW,), jnp.float32),
        pltpu.VMEM((BPW,), jnp.float32),
        pltpu.VMEM((BPW,), jnp.float32),
        pltpu.SemaphoreType.DMA,
    ],
)


@jax.jit
def kernel(i_vecs, j_vecs, ui, uj, bi, bj):
    return _glove_call(i_vecs, j_vecs, ui, uj,
                       bi.reshape(VOCAB), bj.reshape(VOCAB))
